# Initial kernel scaffold; baseline (speedup 1.0000x reference)
import jax
import jax.numpy as jnp
from jax.experimental import pallas as pl


def kernel(x, Wg_proj, bg, expert_emb, W_gate, W_experts, b_experts):
    raise NotImplementedError("write your pallas kernel here")


# speedup vs baseline: 18.4358x; 18.4358x over previous
import functools

import jax
import jax.numpy as jnp
from jax.experimental import pallas as pl
from jax.experimental.pallas import tpu as pltpu

import numpy as np

_SIGN = np.int32(-2147483648)
_LOW = np.int32(0x7FFFFFFF)


def _median_body(x_ref, med_ref):
    xb = x_ref[...]
    rows, s = xb.shape
    k = s // 2 - 1
    u = jax.lax.bitcast_convert_type(xb, jnp.int32)
    keys = jnp.where(u >= 0, u, u ^ _LOW)

    prefix = jnp.zeros((rows, 1), jnp.int32)
    for bit in range(31, -1, -1):
        bitval = jnp.int32(-2147483648) if bit == 31 else jnp.int32(1 << bit)
        cand_u = prefix | bitval
        cand_s = cand_u ^ _SIGN
        cnt = jnp.sum((keys < cand_s).astype(jnp.int32), axis=1, keepdims=True)
        prefix = jnp.where(cnt <= k, cand_u, prefix)
    m1_s = prefix ^ _SIGN
    cnt_le = jnp.sum((keys <= m1_s).astype(jnp.int32), axis=1, keepdims=True)
    bigger = jnp.where(keys > m1_s, keys, jnp.int32(2147483647))
    m2_s = jnp.min(bigger, axis=1, keepdims=True)
    m2_s = jnp.where(cnt_le >= k + 2, m1_s, m2_s)
    f1 = jax.lax.bitcast_convert_type(
        jnp.where(m1_s >= 0, m1_s, m1_s ^ _LOW), jnp.float32)
    f2 = jax.lax.bitcast_convert_type(
        jnp.where(m2_s >= 0, m2_s, m2_s ^ _LOW), jnp.float32)
    med_ref[...] = 0.5 * (f1 + f2)


def _gating_body(logits_ref, gates_ref):
    logits = logits_ref[...]
    e = logits.shape[1]
    cols = jax.lax.broadcasted_iota(jnp.int32, logits.shape, 1)
    v1 = jnp.max(logits, axis=1, keepdims=True)
    i1 = jnp.min(jnp.where(logits == v1, cols, e), axis=1, keepdims=True)
    oh1 = cols == i1
    masked = jnp.where(oh1, -jnp.inf, logits)
    v2 = jnp.max(masked, axis=1, keepdims=True)
    i2 = jnp.min(jnp.where(masked == v2, cols, e), axis=1, keepdims=True)
    oh2 = cols == i2
    e2 = jnp.exp(v2 - v1)
    g1 = 1.0 / (1.0 + e2)
    g2 = e2 / (1.0 + e2)
    gates_ref[...] = jnp.where(oh1, g1, 0.0) + jnp.where(oh2, g2, 0.0)


def _expert_body(x_ref, g_ref, med_ref, w_ref, b_ref, out_ref, wsum_ref):
    @pl.when(pl.program_id(0) == 0)
    def _():
        wsum_ref[...] = jnp.sum(w_ref[...], axis=1)

    g = g_ref[...]
    acc = jnp.dot(g, b_ref[...], preferred_element_type=jnp.float32)
    acc = acc - jnp.dot(g * med_ref[...], wsum_ref[...],
                        preferred_element_type=jnp.float32)
    xb = x_ref[...]
    ne = g.shape[1]
    for ei in range(ne):
        y = jnp.dot(xb, w_ref[ei], preferred_element_type=jnp.float32)
        acc = acc + g[:, ei:ei + 1] * y
    out_ref[...] = acc


@functools.partial(jax.jit, static_argnames=())
def kernel(x, Wg_proj, bg, expert_emb, W_gate, W_experts, b_experts):
    B, S, V = x.shape
    E, _, D = W_experts.shape
    rows = B * V
    xt = jnp.swapaxes(x, 1, 2).reshape(rows, S)

    tile = 512 if rows % 512 == 0 else rows
    grid = rows // tile
    med = pl.pallas_call(
        _median_body,
        grid=(grid,),
        in_specs=[pl.BlockSpec((tile, S), lambda i: (i, 0))],
        out_specs=pl.BlockSpec((tile, 1), lambda i: (i, 0)),
        out_shape=jax.ShapeDtypeStruct((rows, 1), jnp.float32),
    )(xt)

    context = (x[:, -1, :] - med.reshape(B, V)) @ Wg_proj + bg
    ctx_part = context @ W_gate[:D, :]
    emb_part = jnp.sum(expert_emb * W_gate[D:, :].T, axis=1)
    logits = ctx_part + emb_part[None, :]

    gates = pl.pallas_call(
        _gating_body,
        in_specs=[pl.BlockSpec((B, E), lambda: (0, 0))],
        out_specs=pl.BlockSpec((B, E), lambda: (0, 0)),
        out_shape=jax.ShapeDtypeStruct((B, E), jnp.float32),
    )(logits)

    grow = jnp.repeat(gates, V, axis=0)
    out_rows = pl.pallas_call(
        _expert_body,
        grid=(grid,),
        in_specs=[pl.BlockSpec((tile, S), lambda i: (i, 0)),
                  pl.BlockSpec((tile, E), lambda i: (i, 0)),
                  pl.BlockSpec((tile, 1), lambda i: (i, 0)),
                  pl.BlockSpec((E, S, D), lambda i: (0, 0, 0)),
                  pl.BlockSpec((E, D), lambda i: (0, 0))],
        out_specs=pl.BlockSpec((tile, D), lambda i: (i, 0)),
        out_shape=jax.ShapeDtypeStruct((rows, D), jnp.float32),
        scratch_shapes=[pltpu.VMEM((E, D), jnp.float32)],
    )(xt, grow, med, W_experts, b_experts)

    return out_rows.reshape(B, V, D).swapaxes(1, 2)


# speedup vs baseline: 18.5047x; 1.0037x over previous
import functools

import jax
import jax.numpy as jnp
from jax.experimental import pallas as pl
from jax.experimental.pallas import tpu as pltpu

import numpy as np

_SIGN = np.int32(-2147483648)
_LOW = np.int32(0x7FFFFFFF)


def _median_body(x_ref, med_ref):
    xb = x_ref[...]
    rows, s = xb.shape
    k = s // 2 - 1
    u = jax.lax.bitcast_convert_type(xb, jnp.int32)
    keys = jnp.where(u >= 0, u, u ^ _LOW)

    prefix = jnp.zeros((rows, 1), jnp.int32)
    for bit in range(31, -1, -1):
        bitval = jnp.int32(-2147483648) if bit == 31 else jnp.int32(1 << bit)
        cand_u = prefix | bitval
        cand_s = cand_u ^ _SIGN
        cnt = jnp.sum((keys < cand_s).astype(jnp.int32), axis=1, keepdims=True)
        prefix = jnp.where(cnt <= k, cand_u, prefix)
    m1_s = prefix ^ _SIGN
    cnt_le = jnp.sum((keys <= m1_s).astype(jnp.int32), axis=1, keepdims=True)
    bigger = jnp.where(keys > m1_s, keys, jnp.int32(2147483647))
    m2_s = jnp.min(bigger, axis=1, keepdims=True)
    m2_s = jnp.where(cnt_le >= k + 2, m1_s, m2_s)
    f1 = jax.lax.bitcast_convert_type(
        jnp.where(m1_s >= 0, m1_s, m1_s ^ _LOW), jnp.float32)
    f2 = jax.lax.bitcast_convert_type(
        jnp.where(m2_s >= 0, m2_s, m2_s ^ _LOW), jnp.float32)
    med_ref[...] = 0.5 * (f1 + f2)


def _gating_body(logits_ref, gates_ref):
    logits = logits_ref[...]
    e = logits.shape[1]
    cols = jax.lax.broadcasted_iota(jnp.int32, logits.shape, 1)
    v1 = jnp.max(logits, axis=1, keepdims=True)
    i1 = jnp.min(jnp.where(logits == v1, cols, e), axis=1, keepdims=True)
    oh1 = cols == i1
    masked = jnp.where(oh1, -jnp.inf, logits)
    v2 = jnp.max(masked, axis=1, keepdims=True)
    i2 = jnp.min(jnp.where(masked == v2, cols, e), axis=1, keepdims=True)
    oh2 = cols == i2
    e2 = jnp.exp(v2 - v1)
    g1 = 1.0 / (1.0 + e2)
    g2 = e2 / (1.0 + e2)
    gates_ref[...] = jnp.where(oh1, g1, 0.0) + jnp.where(oh2, g2, 0.0)


def _expert_body(x_ref, g_ref, med_ref, w_ref, b_ref, out_ref, wsum_ref,
                 wb_ref):
    ne, s, d = w_ref.shape
    @pl.when(pl.program_id(0) == 0)
    def _():
        wsum_ref[...] = jnp.sum(w_ref[...], axis=1)
        wb_ref[...] = w_ref[...].reshape(ne * s, d).astype(jnp.bfloat16)

    g = g_ref[...]
    acc = jnp.dot(g, b_ref[...], preferred_element_type=jnp.float32)
    acc = acc - jnp.dot(g * med_ref[...], wsum_ref[...],
                        preferred_element_type=jnp.float32)
    xb = x_ref[...].astype(jnp.bfloat16)
    g16 = g.astype(jnp.bfloat16)
    xs = jnp.concatenate([xb * g16[:, e:e + 1] for e in range(ne)], axis=1)
    acc = acc + jnp.dot(xs, wb_ref[...], preferred_element_type=jnp.float32)
    out_ref[...] = acc


@functools.partial(jax.jit, static_argnames=())
def kernel(x, Wg_proj, bg, expert_emb, W_gate, W_experts, b_experts):
    B, S, V = x.shape
    E, _, D = W_experts.shape
    rows = B * V
    xt = jnp.swapaxes(x, 1, 2).reshape(rows, S)

    tile = 512 if rows % 512 == 0 else rows
    grid = rows // tile
    med = pl.pallas_call(
        _median_body,
        grid=(grid,),
        in_specs=[pl.BlockSpec((tile, S), lambda i: (i, 0))],
        out_specs=pl.BlockSpec((tile, 1), lambda i: (i, 0)),
        out_shape=jax.ShapeDtypeStruct((rows, 1), jnp.float32),
    )(xt)

    context = (x[:, -1, :] - med.reshape(B, V)) @ Wg_proj + bg
    ctx_part = context @ W_gate[:D, :]
    emb_part = jnp.sum(expert_emb * W_gate[D:, :].T, axis=1)
    logits = ctx_part + emb_part[None, :]

    gates = pl.pallas_call(
        _gating_body,
        in_specs=[pl.BlockSpec((B, E), lambda: (0, 0))],
        out_specs=pl.BlockSpec((B, E), lambda: (0, 0)),
        out_shape=jax.ShapeDtypeStruct((B, E), jnp.float32),
    )(logits)

    grow = jnp.repeat(gates, V, axis=0)
    out_rows = pl.pallas_call(
        _expert_body,
        grid=(grid,),
        in_specs=[pl.BlockSpec((tile, S), lambda i: (i, 0)),
                  pl.BlockSpec((tile, E), lambda i: (i, 0)),
                  pl.BlockSpec((tile, 1), lambda i: (i, 0)),
                  pl.BlockSpec((E, S, D), lambda i: (0, 0, 0)),
                  pl.BlockSpec((E, D), lambda i: (0, 0))],
        out_specs=pl.BlockSpec((tile, D), lambda i: (i, 0)),
        out_shape=jax.ShapeDtypeStruct((rows, D), jnp.float32),
        scratch_shapes=[pltpu.VMEM((E, D), jnp.float32),
                        pltpu.VMEM((E * S, D), jnp.bfloat16)],
    )(xt, grow, med, W_experts, b_experts)

    return out_rows.reshape(B, V, D).swapaxes(1, 2)


# speedup vs baseline: 21.9558x; 1.1865x over previous
import functools

import jax
import jax.numpy as jnp
from jax.experimental import pallas as pl
from jax.experimental.pallas import tpu as pltpu

import numpy as np

_SIGN = np.int32(-2147483648)
_LOW = np.int32(0x7FFFFFFF)


def _median_body(x_ref, med_ref):
    xb = x_ref[...]
    s, r = xb.shape
    k = s // 2 - 1
    u = jax.lax.bitcast_convert_type(xb, jnp.int32)
    keys = jnp.where(u >= 0, u, u ^ _LOW)

    prefix = jnp.zeros((1, r), jnp.int32)
    for bit in range(31, -1, -1):
        bitval = jnp.int32(-2147483648) if bit == 31 else jnp.int32(1 << bit)
        cand_u = prefix | bitval
        cand_s = cand_u ^ _SIGN
        cnt = jnp.sum((keys < cand_s).astype(jnp.int32), axis=0, keepdims=True)
        prefix = jnp.where(cnt <= k, cand_u, prefix)
    m1_s = prefix ^ _SIGN
    cnt_le = jnp.sum((keys <= m1_s).astype(jnp.int32), axis=0, keepdims=True)
    bigger = jnp.where(keys > m1_s, keys, jnp.int32(2147483647))
    m2_s = jnp.min(bigger, axis=0, keepdims=True)
    m2_s = jnp.where(cnt_le >= k + 2, m1_s, m2_s)
    f1 = jax.lax.bitcast_convert_type(
        jnp.where(m1_s >= 0, m1_s, m1_s ^ _LOW), jnp.float32)
    f2 = jax.lax.bitcast_convert_type(
        jnp.where(m2_s >= 0, m2_s, m2_s ^ _LOW), jnp.float32)
    med_ref[...] = (0.5 * (f1 + f2))[None]


def _gating_body(logits_ref, gates_ref):
    logits = logits_ref[...]
    e = logits.shape[1]
    cols = jax.lax.broadcasted_iota(jnp.int32, logits.shape, 1)
    v1 = jnp.max(logits, axis=1, keepdims=True)
    i1 = jnp.min(jnp.where(logits == v1, cols, e), axis=1, keepdims=True)
    oh1 = cols == i1
    masked = jnp.where(oh1, -jnp.inf, logits)
    v2 = jnp.max(masked, axis=1, keepdims=True)
    i2 = jnp.min(jnp.where(masked == v2, cols, e), axis=1, keepdims=True)
    oh2 = cols == i2
    e2 = jnp.exp(v2 - v1)
    g1 = 1.0 / (1.0 + e2)
    g2 = e2 / (1.0 + e2)
    gates_ref[...] = jnp.where(oh1, g1, 0.0) + jnp.where(oh2, g2, 0.0)


def _expert_body(x_ref, g_ref, med_ref, w_ref, b_ref, out_ref, wsum_ref,
                 wb_ref):
    ne, s, d = w_ref.shape
    @pl.when(pl.program_id(0) == 0)
    def _():
        wsum_ref[...] = jnp.sum(w_ref[...], axis=1)
        wb_ref[...] = w_ref[...].reshape(ne * s, d).astype(jnp.bfloat16)

    g = g_ref[...]
    med = med_ref[0]
    dn0 = (((0,), (0,)), ((), ()))
    acc = jax.lax.dot_general(g, b_ref[...], dn0,
                              preferred_element_type=jnp.float32)
    acc = acc - jax.lax.dot_general(g * med, wsum_ref[...], dn0,
                                    preferred_element_type=jnp.float32)
    xb = x_ref[...].astype(jnp.bfloat16)
    g16 = g.astype(jnp.bfloat16)
    xs = jnp.concatenate([xb * g16[e:e + 1] for e in range(ne)], axis=0)
    acc = acc + jax.lax.dot_general(xs, wb_ref[...], dn0,
                                    preferred_element_type=jnp.float32)
    out_ref[...] = acc


@functools.partial(jax.jit, static_argnames=())
def kernel(x, Wg_proj, bg, expert_emb, W_gate, W_experts, b_experts):
    B, S, V = x.shape
    E, _, D = W_experts.shape
    rows = B * V
    xt = jnp.transpose(x, (1, 0, 2)).reshape(S, rows)

    tile = 512 if rows % 512 == 0 else rows
    grid = rows // tile
    med2d = pl.pallas_call(
        _median_body,
        grid=(grid,),
        in_specs=[pl.BlockSpec((S, tile), lambda i: (0, i))],
        out_specs=pl.BlockSpec((1, 1, tile), lambda i: (i, 0, 0)),
        out_shape=jax.ShapeDtypeStruct((grid, 1, tile), jnp.float32),
    )(xt)
    med_row = med2d.reshape(1, rows)

    context = (x[:, -1, :] - med_row.reshape(B, V)) @ Wg_proj + bg
    ctx_part = context @ W_gate[:D, :]
    emb_part = jnp.sum(expert_emb * W_gate[D:, :].T, axis=1)
    logits = ctx_part + emb_part[None, :]

    gates = pl.pallas_call(
        _gating_body,
        in_specs=[pl.BlockSpec((B, E), lambda: (0, 0))],
        out_specs=pl.BlockSpec((B, E), lambda: (0, 0)),
        out_shape=jax.ShapeDtypeStruct((B, E), jnp.float32),
    )(logits)

    g_t = jnp.repeat(gates, V, axis=0).T
    med3d = med2d.reshape(grid, 1, tile)
    out_rows = pl.pallas_call(
        _expert_body,
        grid=(grid,),
        in_specs=[pl.BlockSpec((S, tile), lambda i: (0, i)),
                  pl.BlockSpec((E, tile), lambda i: (0, i)),
                  pl.BlockSpec((1, 1, tile), lambda i: (i, 0, 0)),
                  pl.BlockSpec((E, S, D), lambda i: (0, 0, 0)),
                  pl.BlockSpec((E, D), lambda i: (0, 0))],
        out_specs=pl.BlockSpec((tile, D), lambda i: (i, 0)),
        out_shape=jax.ShapeDtypeStruct((rows, D), jnp.float32),
        scratch_shapes=[pltpu.VMEM((E, D), jnp.float32),
                        pltpu.VMEM((E * S, D), jnp.bfloat16)],
    )(xt, g_t, med3d, W_experts, b_experts)

    return out_rows.reshape(B, V, D).swapaxes(1, 2)


# speedup vs baseline: 22.1673x; 1.0096x over previous
import functools

import jax
import jax.numpy as jnp
from jax.experimental import pallas as pl
from jax.experimental.pallas import tpu as pltpu

import numpy as np

_SIGN = np.int32(-2147483648)
_LOW = np.int32(0x7FFFFFFF)


def _median_body(x_ref, med_ref):
    xb = x_ref[...]
    s, r = xb.shape
    k = s // 2 - 1
    u = jax.lax.bitcast_convert_type(xb, jnp.int32)
    keys = jnp.where(u >= 0, u, u ^ _LOW)

    prefix = jnp.zeros((1, r), jnp.int32)
    for bit in range(31, -1, -1):
        bitval = jnp.int32(-2147483648) if bit == 31 else jnp.int32(1 << bit)
        cand_u = prefix | bitval
        cand_s = cand_u ^ _SIGN
        cnt = jnp.sum((keys < cand_s).astype(jnp.int32), axis=0, keepdims=True)
        prefix = jnp.where(cnt <= k, cand_u, prefix)
    m1_s = prefix ^ _SIGN
    cnt_le = jnp.sum((keys <= m1_s).astype(jnp.int32), axis=0, keepdims=True)
    bigger = jnp.where(keys > m1_s, keys, jnp.int32(2147483647))
    m2_s = jnp.min(bigger, axis=0, keepdims=True)
    m2_s = jnp.where(cnt_le >= k + 2, m1_s, m2_s)
    f1 = jax.lax.bitcast_convert_type(
        jnp.where(m1_s >= 0, m1_s, m1_s ^ _LOW), jnp.float32)
    f2 = jax.lax.bitcast_convert_type(
        jnp.where(m2_s >= 0, m2_s, m2_s ^ _LOW), jnp.float32)
    med_ref[...] = (0.5 * (f1 + f2))[None]


def _gating_body(logits_ref, gates_ref):
    logits = logits_ref[...]
    e = logits.shape[1]
    cols = jax.lax.broadcasted_iota(jnp.int32, logits.shape, 1)
    v1 = jnp.max(logits, axis=1, keepdims=True)
    i1 = jnp.min(jnp.where(logits == v1, cols, e), axis=1, keepdims=True)
    oh1 = cols == i1
    masked = jnp.where(oh1, -jnp.inf, logits)
    v2 = jnp.max(masked, axis=1, keepdims=True)
    i2 = jnp.min(jnp.where(masked == v2, cols, e), axis=1, keepdims=True)
    oh2 = cols == i2
    e2 = jnp.exp(v2 - v1)
    g1 = 1.0 / (1.0 + e2)
    g2 = e2 / (1.0 + e2)
    gates_ref[...] = jnp.where(oh1, g1, 0.0) + jnp.where(oh2, g2, 0.0)


def _expert_body(x_ref, g_ref, med_ref, w_ref, b_ref, out_ref, wsum_ref,
                 wb_ref):
    ne, s, d = w_ref.shape
    @pl.when(pl.program_id(0) == 0)
    def _():
        wsum_ref[...] = jnp.sum(w_ref[...], axis=1)
        wb_ref[...] = w_ref[...].reshape(ne * s, d).astype(jnp.bfloat16)

    g = g_ref[...]
    med = med_ref[0]
    dn0 = (((0,), (0,)), ((), ()))
    acc = jax.lax.dot_general(g, b_ref[...], dn0,
                              preferred_element_type=jnp.float32)
    acc = acc - jax.lax.dot_general(g * med, wsum_ref[...], dn0,
                                    preferred_element_type=jnp.float32)
    xb = x_ref[...].astype(jnp.bfloat16)
    g16 = g.astype(jnp.bfloat16)
    xs = jnp.concatenate([xb * g16[e:e + 1] for e in range(ne)], axis=0)
    acc = acc + jax.lax.dot_general(xs, wb_ref[...], dn0,
                                    preferred_element_type=jnp.float32)
    out_ref[...] = acc


@functools.partial(jax.jit, static_argnames=())
def kernel(x, Wg_proj, bg, expert_emb, W_gate, W_experts, b_experts):
    B, S, V = x.shape
    E, _, D = W_experts.shape
    rows = B * V
    xt = jnp.transpose(x, (1, 0, 2)).reshape(S, rows)

    tile = 512 if rows % 512 == 0 else rows
    grid = rows // tile
    med2d = pl.pallas_call(
        _median_body,
        grid=(grid,),
        in_specs=[pl.BlockSpec((S, tile), lambda i: (0, i))],
        out_specs=pl.BlockSpec((1, 1, tile), lambda i: (i, 0, 0)),
        out_shape=jax.ShapeDtypeStruct((grid, 1, tile), jnp.float32),
    )(xt)
    med_row = med2d.reshape(1, rows)

    context = (x[:, -1, :] - med_row.reshape(B, V)) @ Wg_proj + bg
    ctx_part = context @ W_gate[:D, :]
    emb_part = jnp.sum(expert_emb * W_gate[D:, :].T, axis=1)
    logits = ctx_part + emb_part[None, :]

    gates = pl.pallas_call(
        _gating_body,
        in_specs=[pl.BlockSpec((B, E), lambda: (0, 0))],
        out_specs=pl.BlockSpec((B, E), lambda: (0, 0)),
        out_shape=jax.ShapeDtypeStruct((B, E), jnp.float32),
    )(logits)

    g_t = jnp.repeat(gates, V, axis=0).T
    etile = 1024 if rows % 1024 == 0 else tile
    egrid = rows // etile
    med3d = med2d.reshape(egrid, 1, etile)
    out_rows = pl.pallas_call(
        _expert_body,
        grid=(egrid,),
        in_specs=[pl.BlockSpec((S, etile), lambda i: (0, i)),
                  pl.BlockSpec((E, etile), lambda i: (0, i)),
                  pl.BlockSpec((1, 1, etile), lambda i: (i, 0, 0)),
                  pl.BlockSpec((E, S, D), lambda i: (0, 0, 0)),
                  pl.BlockSpec((E, D), lambda i: (0, 0))],
        out_specs=pl.BlockSpec((etile, D), lambda i: (i, 0)),
        out_shape=jax.ShapeDtypeStruct((rows, D), jnp.float32),
        scratch_shapes=[pltpu.VMEM((E, D), jnp.float32),
                        pltpu.VMEM((E * S, D), jnp.bfloat16)],
    )(xt, g_t, med3d, W_experts, b_experts)

    return out_rows.reshape(B, V, D).swapaxes(1, 2)


# speedup vs baseline: 24.2435x; 1.0937x over previous
import functools

import jax
import jax.numpy as jnp
from jax.experimental import pallas as pl
from jax.experimental.pallas import tpu as pltpu

import numpy as np

_SIGN = np.int32(-2147483648)
_LOW = np.int32(0x7FFFFFFF)


def _median_body(x_ref, med_ref):
    xb = jnp.transpose(x_ref[...])
    s, r = xb.shape
    k = s // 2 - 1
    u = jax.lax.bitcast_convert_type(xb, jnp.int32)
    keys = jnp.where(u >= 0, u, u ^ _LOW)

    prefix = jnp.zeros((1, r), jnp.int32)
    for bit in range(31, -1, -1):
        bitval = jnp.int32(-2147483648) if bit == 31 else jnp.int32(1 << bit)
        cand_u = prefix | bitval
        cand_s = cand_u ^ _SIGN
        cnt = jnp.sum((keys < cand_s).astype(jnp.int32), axis=0, keepdims=True)
        prefix = jnp.where(cnt <= k, cand_u, prefix)
    m1_s = prefix ^ _SIGN
    cnt_le = jnp.sum((keys <= m1_s).astype(jnp.int32), axis=0, keepdims=True)
    bigger = jnp.where(keys > m1_s, keys, jnp.int32(2147483647))
    m2_s = jnp.min(bigger, axis=0, keepdims=True)
    m2_s = jnp.where(cnt_le >= k + 2, m1_s, m2_s)
    f1 = jax.lax.bitcast_convert_type(
        jnp.where(m1_s >= 0, m1_s, m1_s ^ _LOW), jnp.float32)
    f2 = jax.lax.bitcast_convert_type(
        jnp.where(m2_s >= 0, m2_s, m2_s ^ _LOW), jnp.float32)
    med_ref[...] = (0.5 * (f1 + f2))[None]


def _gating_body(logits_ref, gates_ref):
    logits = logits_ref[...]
    e = logits.shape[1]
    cols = jax.lax.broadcasted_iota(jnp.int32, logits.shape, 1)
    v1 = jnp.max(logits, axis=1, keepdims=True)
    i1 = jnp.min(jnp.where(logits == v1, cols, e), axis=1, keepdims=True)
    oh1 = cols == i1
    masked = jnp.where(oh1, -jnp.inf, logits)
    v2 = jnp.max(masked, axis=1, keepdims=True)
    i2 = jnp.min(jnp.where(masked == v2, cols, e), axis=1, keepdims=True)
    oh2 = cols == i2
    e2 = jnp.exp(v2 - v1)
    g1 = 1.0 / (1.0 + e2)
    g2 = e2 / (1.0 + e2)
    gates_ref[...] = jnp.where(oh1, g1, 0.0) + jnp.where(oh2, g2, 0.0)


def _expert_body(x_ref, g_ref, med_ref, w_ref, b_ref, out_ref, wsum_ref,
                 wb_ref):
    ne, s, d = w_ref.shape
    @pl.when(pl.program_id(0) == 0)
    def _():
        wsum_ref[...] = jnp.sum(w_ref[...], axis=1)
        wb_ref[...] = w_ref[...].reshape(ne * s, d).astype(jnp.bfloat16)

    g = g_ref[...]
    med = med_ref[...]
    acc = jnp.dot(g, b_ref[...], preferred_element_type=jnp.float32)
    acc = acc - jnp.dot(g * med, wsum_ref[...],
                        preferred_element_type=jnp.float32)
    xb = x_ref[...].astype(jnp.bfloat16)
    g16 = g.astype(jnp.bfloat16)
    xs = jnp.concatenate([xb * g16[:, e:e + 1] for e in range(ne)], axis=1)
    acc = acc + jnp.dot(xs, wb_ref[...], preferred_element_type=jnp.float32)
    out_ref[...] = acc


@functools.partial(jax.jit, static_argnames=())
def kernel(x, Wg_proj, bg, expert_emb, W_gate, W_experts, b_experts):
    B, S, V = x.shape
    E, _, D = W_experts.shape
    rows = B * V
    xt = jnp.swapaxes(x, 1, 2).reshape(rows, S)

    tile = 512 if rows % 512 == 0 else rows
    grid = rows // tile
    med2d = pl.pallas_call(
        _median_body,
        grid=(grid,),
        in_specs=[pl.BlockSpec((tile, S), lambda i: (i, 0))],
        out_specs=pl.BlockSpec((1, 1, tile), lambda i: (i, 0, 0)),
        out_shape=jax.ShapeDtypeStruct((grid, 1, tile), jnp.float32),
    )(xt)
    med_row = med2d.reshape(1, rows)

    context = (x[:, -1, :] - med_row.reshape(B, V)) @ Wg_proj + bg
    ctx_part = context @ W_gate[:D, :]
    emb_part = jnp.sum(expert_emb * W_gate[D:, :].T, axis=1)
    logits = ctx_part + emb_part[None, :]

    gates = pl.pallas_call(
        _gating_body,
        in_specs=[pl.BlockSpec((B, E), lambda: (0, 0))],
        out_specs=pl.BlockSpec((B, E), lambda: (0, 0)),
        out_shape=jax.ShapeDtypeStruct((B, E), jnp.float32),
    )(logits)

    g_rows = jnp.repeat(gates, V, axis=0)
    med_col = med2d.reshape(rows, 1)
    etile = 1024 if rows % 1024 == 0 else tile
    egrid = rows // etile
    out_rows = pl.pallas_call(
        _expert_body,
        grid=(egrid,),
        in_specs=[pl.BlockSpec((etile, S), lambda i: (i, 0)),
                  pl.BlockSpec((etile, E), lambda i: (i, 0)),
                  pl.BlockSpec((etile, 1), lambda i: (i, 0)),
                  pl.BlockSpec((E, S, D), lambda i: (0, 0, 0)),
                  pl.BlockSpec((E, D), lambda i: (0, 0))],
        out_specs=pl.BlockSpec((etile, D), lambda i: (i, 0)),
        out_shape=jax.ShapeDtypeStruct((rows, D), jnp.float32),
        scratch_shapes=[pltpu.VMEM((E, D), jnp.float32),
                        pltpu.VMEM((E * S, D), jnp.bfloat16)],
    )(xt, g_rows, med_col, W_experts, b_experts)

    return out_rows.reshape(B, V, D).swapaxes(1, 2)


# speedup vs baseline: 27.2127x; 1.1225x over previous
import functools

import jax
import jax.numpy as jnp
from jax.experimental import pallas as pl
from jax.experimental.pallas import tpu as pltpu

import numpy as np

_SIGN = np.int32(-2147483648)
_LOW = np.int32(0x7FFFFFFF)


def _sum16(m):
    while m.shape[0] > 1:
        h = m.shape[0] // 2
        m = m[:h] + m[h:]
    return m.astype(jnp.int32)


def _median_body(x_ref, med_ref):
    xb = jnp.transpose(x_ref[...])
    s, r = xb.shape
    k = s // 2 - 1
    u = jax.lax.bitcast_convert_type(xb, jnp.int32)
    keys = jnp.where(u >= 0, u, u ^ _LOW)

    khi = (keys >> 16).astype(jnp.int16)
    prefix_hi = jnp.zeros((1, r), jnp.int32)
    for j in range(15, -1, -1):
        cand_u = prefix_hi | jnp.int32(1 << j)
        cand_s16 = (cand_u - 32768).astype(jnp.int16)
        cnt = _sum16((khi < cand_s16).astype(jnp.int16))
        prefix_hi = jnp.where(cnt <= k, cand_u, prefix_hi)
    phi_s16 = (prefix_hi - 32768).astype(jnp.int16)

    lo_p = ((keys & jnp.int32(0xFFFF)) - 32768).astype(jnp.int16)
    below = khi < phi_s16
    above = khi > phi_s16
    lo2 = jnp.where(below, jnp.int16(-32768),
                    jnp.where(above, jnp.int16(32767), lo_p))
    prefix_lo = jnp.zeros((1, r), jnp.int32)
    for j in range(15, -1, -1):
        cand_u = prefix_lo | jnp.int32(1 << j)
        cand_s16 = (cand_u - 32768).astype(jnp.int16)
        cnt = _sum16((lo2 < cand_s16).astype(jnp.int16))
        prefix_lo = jnp.where(cnt <= k, cand_u, prefix_lo)

    m1_s = ((prefix_hi << 16) | prefix_lo) ^ _SIGN
    cnt_le = jnp.sum((keys <= m1_s).astype(jnp.int32), axis=0, keepdims=True)
    bigger = jnp.where(keys > m1_s, keys, jnp.int32(2147483647))
    m2_s = jnp.min(bigger, axis=0, keepdims=True)
    m2_s = jnp.where(cnt_le >= k + 2, m1_s, m2_s)
    f1 = jax.lax.bitcast_convert_type(
        jnp.where(m1_s >= 0, m1_s, m1_s ^ _LOW), jnp.float32)
    f2 = jax.lax.bitcast_convert_type(
        jnp.where(m2_s >= 0, m2_s, m2_s ^ _LOW), jnp.float32)
    med_ref[...] = (0.5 * (f1 + f2))[None]


def _gating_body(logits_ref, gates_ref):
    logits = logits_ref[...]
    e = logits.shape[1]
    cols = jax.lax.broadcasted_iota(jnp.int32, logits.shape, 1)
    v1 = jnp.max(logits, axis=1, keepdims=True)
    i1 = jnp.min(jnp.where(logits == v1, cols, e), axis=1, keepdims=True)
    oh1 = cols == i1
    masked = jnp.where(oh1, -jnp.inf, logits)
    v2 = jnp.max(masked, axis=1, keepdims=True)
    i2 = jnp.min(jnp.where(masked == v2, cols, e), axis=1, keepdims=True)
    oh2 = cols == i2
    e2 = jnp.exp(v2 - v1)
    g1 = 1.0 / (1.0 + e2)
    g2 = e2 / (1.0 + e2)
    gates_ref[...] = jnp.where(oh1, g1, 0.0) + jnp.where(oh2, g2, 0.0)


def _expert_body(x_ref, g_ref, med_ref, w_ref, b_ref, out_ref, wsum_ref,
                 wb_ref):
    ne, s, d = w_ref.shape
    @pl.when(pl.program_id(0) == 0)
    def _():
        wsum_ref[...] = jnp.sum(w_ref[...], axis=1)
        wb_ref[...] = w_ref[...].reshape(ne * s, d).astype(jnp.bfloat16)

    g = g_ref[...]
    med = med_ref[...]
    acc = jnp.dot(g, b_ref[...], preferred_element_type=jnp.float32)
    acc = acc - jnp.dot(g * med, wsum_ref[...],
                        preferred_element_type=jnp.float32)
    xb = x_ref[...].astype(jnp.bfloat16)
    g16 = g.astype(jnp.bfloat16)
    xs = jnp.concatenate([xb * g16[:, e:e + 1] for e in range(ne)], axis=1)
    acc = acc + jnp.dot(xs, wb_ref[...], preferred_element_type=jnp.float32)
    out_ref[...] = acc


@functools.partial(jax.jit, static_argnames=())
def kernel(x, Wg_proj, bg, expert_emb, W_gate, W_experts, b_experts):
    B, S, V = x.shape
    E, _, D = W_experts.shape
    rows = B * V
    xt = jnp.swapaxes(x, 1, 2).reshape(rows, S)

    tile = 512 if rows % 512 == 0 else rows
    grid = rows // tile
    med2d = pl.pallas_call(
        _median_body,
        grid=(grid,),
        in_specs=[pl.BlockSpec((tile, S), lambda i: (i, 0))],
        out_specs=pl.BlockSpec((1, 1, tile), lambda i: (i, 0, 0)),
        out_shape=jax.ShapeDtypeStruct((grid, 1, tile), jnp.float32),
    )(xt)
    med_row = med2d.reshape(1, rows)

    context = (x[:, -1, :] - med_row.reshape(B, V)) @ Wg_proj + bg
    ctx_part = context @ W_gate[:D, :]
    emb_part = jnp.sum(expert_emb * W_gate[D:, :].T, axis=1)
    logits = ctx_part + emb_part[None, :]

    gates = pl.pallas_call(
        _gating_body,
        in_specs=[pl.BlockSpec((B, E), lambda: (0, 0))],
        out_specs=pl.BlockSpec((B, E), lambda: (0, 0)),
        out_shape=jax.ShapeDtypeStruct((B, E), jnp.float32),
    )(logits)

    g_rows = jnp.repeat(gates, V, axis=0)
    med_col = med2d.reshape(rows, 1)
    etile = 1024 if rows % 1024 == 0 else tile
    egrid = rows // etile
    out_rows = pl.pallas_call(
        _expert_body,
        grid=(egrid,),
        in_specs=[pl.BlockSpec((etile, S), lambda i: (i, 0)),
                  pl.BlockSpec((etile, E), lambda i: (i, 0)),
                  pl.BlockSpec((etile, 1), lambda i: (i, 0)),
                  pl.BlockSpec((E, S, D), lambda i: (0, 0, 0)),
                  pl.BlockSpec((E, D), lambda i: (0, 0))],
        out_specs=pl.BlockSpec((etile, D), lambda i: (i, 0)),
        out_shape=jax.ShapeDtypeStruct((rows, D), jnp.float32),
        scratch_shapes=[pltpu.VMEM((E, D), jnp.float32),
                        pltpu.VMEM((E * S, D), jnp.bfloat16)],
    )(xt, g_rows, med_col, W_experts, b_experts)

    return out_rows.reshape(B, V, D).swapaxes(1, 2)
